# Initial kernel scaffold; baseline (speedup 1.0000x reference)
import functools

import jax
import jax.numpy as jnp
from jax import lax
from jax.experimental import pallas as pl
from jax.experimental.pallas import tpu as pltpu

B, S, D = 1, 2048, 768
DFF = 3072
E = 8
DH = 768
TOPK = 2
TEMP = 1.0
EPS = 1e-5

T_BLK = 256


def _fused_body(x_ref, ln_g_ref, ln_b_ref, wfc1_ref, bfc1_ref, wfc2_ref,
                bfc2_ref, wr_ref, br_ref, we1_ref, be1_ref, we2_ref,
                be2_ref, out_ref):
    xb = x_ref[...]
    mu = jnp.mean(xb, axis=-1, keepdims=True)
    var = jnp.mean((xb - mu) ** 2, axis=-1, keepdims=True)
    hb = (xb - mu) / jnp.sqrt(var + EPS) * ln_g_ref[...] + ln_b_ref[...]
    hbb = hb.astype(jnp.bfloat16)

    t1 = jnp.dot(hbb, wfc1_ref[...], preferred_element_type=jnp.float32)
    t1 = jax.nn.gelu(t1 + bfc1_ref[...])
    shared = jnp.dot(t1.astype(jnp.bfloat16), wfc2_ref[...],
                     preferred_element_type=jnp.float32) + bfc2_ref[...]

    logits = jnp.dot(hb, wr_ref[...], preferred_element_type=jnp.float32,
                     precision=lax.Precision.HIGHEST) + br_ref[...]
    idx = lax.broadcasted_iota(jnp.int32, (T_BLK, E), 1)
    m1 = jnp.max(logits, axis=-1, keepdims=True)
    i1 = jnp.min(jnp.where(logits == m1, idx, E), axis=-1, keepdims=True)
    mask1 = idx == i1
    rest = jnp.where(mask1, -jnp.inf, logits)
    m2 = jnp.max(rest, axis=-1, keepdims=True)
    i2 = jnp.min(jnp.where(rest == m2, idx, E), axis=-1, keepdims=True)
    mask2 = idx == i2
    w1 = 1.0 / (1.0 + jnp.exp((m2 - m1) / TEMP))
    gates = jnp.where(mask1, w1, 0.0) + jnp.where(mask2, 1.0 - w1, 0.0)

    acc = xb + shared
    for e in range(E):
        a1 = jnp.dot(hbb, we1_ref[e], preferred_element_type=jnp.float32)
        a1 = jax.nn.gelu(a1 + be1_ref[e])
        y = jnp.dot(a1.astype(jnp.bfloat16), we2_ref[e],
                    preferred_element_type=jnp.float32) + be2_ref[e]
        acc = acc + gates[:, e:e + 1] * y
    out_ref[...] = acc


@jax.jit
def kernel(hidden_states, ln_g, ln_b, W_fc1, b_fc1, W_fc2, b_fc2,
           W_router, b_router, We1, be1, We2, be2):
    x = hidden_states.reshape(S, D)
    bf = jnp.bfloat16
    full = lambda shape: pl.BlockSpec(shape, lambda t: (0,) * len(shape))
    out = pl.pallas_call(
        _fused_body,
        grid=(S // T_BLK,),
        in_specs=[
            pl.BlockSpec((T_BLK, D), lambda t: (t, 0)),
            full((1, D)), full((1, D)),
            full((D, DFF)), full((1, DFF)),
            full((DFF, D)), full((1, D)),
            full((D, E)), full((1, E)),
            full((E, D, DH)), full((E, DH)),
            full((E, DH, D)), full((E, D)),
        ],
        out_specs=pl.BlockSpec((T_BLK, D), lambda t: (t, 0)),
        out_shape=jax.ShapeDtypeStruct((S, D), jnp.float32),
        compiler_params=pltpu.CompilerParams(
            dimension_semantics=("arbitrary",),
        ),
    )(
        x, ln_g.reshape(1, D), ln_b.reshape(1, D),
        W_fc1.astype(bf), b_fc1.reshape(1, DFF),
        W_fc2.astype(bf), b_fc2.reshape(1, D),
        W_router, b_router.reshape(1, E),
        We1.astype(bf), be1,
        We2.astype(bf), be2,
    )
    return out.reshape(B, S, D)


# speedup vs baseline: 1.0531x; 1.0531x over previous
import functools

import jax
import jax.numpy as jnp
from jax import lax
from jax.experimental import pallas as pl
from jax.experimental.pallas import tpu as pltpu

B, S, D = 1, 2048, 768
DFF = 3072
E = 8
DH = 768
TOPK = 2
TEMP = 1.0
EPS = 1e-5

T_BLK = 256


def _fused_body(x_ref, ln_g_ref, ln_b_ref, wfc1_ref, bfc1_ref, wfc2_ref,
                bfc2_ref, wr_ref, br_ref, we1_ref, be1_ref, we2_ref,
                be2_ref, out_ref):
    xb = x_ref[...]
    mu = jnp.mean(xb, axis=-1, keepdims=True)
    var = jnp.mean((xb - mu) ** 2, axis=-1, keepdims=True)
    hb = (xb - mu) / jnp.sqrt(var + EPS) * ln_g_ref[...] + ln_b_ref[...]
    hbb = hb.astype(jnp.bfloat16)

    t1 = jnp.dot(hbb, wfc1_ref[...], preferred_element_type=jnp.float32)
    t1 = jax.nn.gelu(t1 + bfc1_ref[...])
    shared = jnp.dot(t1.astype(jnp.bfloat16), wfc2_ref[...],
                     preferred_element_type=jnp.float32) + bfc2_ref[...]

    logits = jnp.dot(hbb, wr_ref[...],
                     preferred_element_type=jnp.float32) + br_ref[...]
    idx = lax.broadcasted_iota(jnp.int32, (T_BLK, E), 1)
    m1 = jnp.max(logits, axis=-1, keepdims=True)
    i1 = jnp.min(jnp.where(logits == m1, idx, E), axis=-1, keepdims=True)
    mask1 = idx == i1
    rest = jnp.where(mask1, -jnp.inf, logits)
    m2 = jnp.max(rest, axis=-1, keepdims=True)
    i2 = jnp.min(jnp.where(rest == m2, idx, E), axis=-1, keepdims=True)
    mask2 = idx == i2
    w1 = 1.0 / (1.0 + jnp.exp((m2 - m1) / TEMP))
    gates = jnp.where(mask1, w1, 0.0) + jnp.where(mask2, 1.0 - w1, 0.0)

    acc = xb + shared
    for e in range(E):
        a1 = jnp.dot(hbb, we1_ref[e], preferred_element_type=jnp.float32)
        a1 = jax.nn.gelu(a1 + be1_ref[e])
        y = jnp.dot(a1.astype(jnp.bfloat16), we2_ref[e],
                    preferred_element_type=jnp.float32) + be2_ref[e]
        acc = acc + gates[:, e:e + 1] * y
    out_ref[...] = acc


@jax.jit
def kernel(hidden_states, ln_g, ln_b, W_fc1, b_fc1, W_fc2, b_fc2,
           W_router, b_router, We1, be1, We2, be2):
    x = hidden_states.reshape(S, D)
    bf = jnp.bfloat16
    full = lambda shape: pl.BlockSpec(shape, lambda t: (0,) * len(shape))
    out = pl.pallas_call(
        _fused_body,
        grid=(S // T_BLK,),
        in_specs=[
            pl.BlockSpec((T_BLK, D), lambda t: (t, 0)),
            full((1, D)), full((1, D)),
            full((D, DFF)), full((1, DFF)),
            full((DFF, D)), full((1, D)),
            full((D, E)), full((1, E)),
            full((E, D, DH)), full((E, DH)),
            full((E, DH, D)), full((E, D)),
        ],
        out_specs=pl.BlockSpec((T_BLK, D), lambda t: (t, 0)),
        out_shape=jax.ShapeDtypeStruct((S, D), jnp.float32),
        compiler_params=pltpu.CompilerParams(
            dimension_semantics=("arbitrary",),
        ),
    )(
        x, ln_g.reshape(1, D), ln_b.reshape(1, D),
        W_fc1.astype(bf), b_fc1.reshape(1, DFF),
        W_fc2.astype(bf), b_fc2.reshape(1, D),
        W_router.astype(bf), b_router.reshape(1, E),
        We1.astype(bf), be1,
        We2.astype(bf), be2,
    )
    return out.reshape(B, S, D)
